# baseline (speedup 1.0000x reference)
import functools

import jax
import jax.numpy as jnp
from jax import lax
from jax.experimental import pallas as pl
from jax.experimental.pallas import tpu as pltpu
from jax.experimental.pallas import tpu_sc as plsc

N_NODES = 10000
N_PAD = 10240
D = 128
N_EDGES = 320000
NC = 2
NS = 16
NW = NC * NS
CHUNK = 80
NCHUNKS = 128
E_TILE = N_EDGES // NW
PAD_TILE = NCHUNKS * CHUNK - E_TILE
ROWS_PER_TILE = N_PAD // NS


def _sc_agg_body(x_hbm, src_hbm, dst_hbm, zero_hbm, out_hbm,
                 acc_sh, rows0, rows1,
                 is0, is1, is2, is3, id0, id1, id2, id3,
                 gsem0, gsem1, isem0, isem1, isem2, isem3):
    c = lax.axis_index("c")
    s = lax.axis_index("s")
    r0 = s * ROWS_PER_TILE
    pltpu.sync_copy(zero_hbm.at[pl.ds(r0, ROWS_PER_TILE)],
                    acc_sh.at[pl.ds(r0, ROWS_PER_TILE)])
    wid = c * NS + s

    rows = (rows0, rows1)
    gsem = (gsem0, gsem1)
    isb = (is0, is1, is2, is3)
    idb = (id0, id1, id2, id3)
    isem = (isem0, isem1, isem2, isem3)

    def issue_idx(n, k):
        pltpu.async_copy(src_hbm.at[wid, n], isb[k], isem[k])
        pltpu.async_copy(dst_hbm.at[wid, n], idb[k], isem[k])

    def wait_idx(k):
        pltpu.make_async_copy(src_hbm.at[wid, 0], isb[k], isem[k]).wait()
        pltpu.make_async_copy(dst_hbm.at[wid, 0], idb[k], isem[k]).wait()

    pltpu.sync_copy(src_hbm.at[wid, 0], is0)
    pltpu.sync_copy(dst_hbm.at[wid, 0], id0)
    issue_idx(1, 1)
    issue_idx(2, 2)
    plsc.subcore_barrier()
    pltpu.async_copy(x_hbm.at[is0], rows0, gsem0)

    def body(t, carry):
        for b in range(4):
            i = 4 * t + b
            rb, rbn = b % 2, (b + 1) % 2
            bn, bp = (b + 1) % 4, (b + 3) % 4
            wait_idx(bn)
            pltpu.async_copy(x_hbm.at[isb[bn]], rows[rbn], gsem[rbn])
            pltpu.make_async_copy(x_hbm.at[isb[b]], rows[rb],
                                  gsem[rb]).wait()
            pltpu.sync_copy(rows[rb], acc_sh.at[idb[b]], add=True)
            nx3 = jnp.minimum(i + 3, NCHUNKS - 1)
            issue_idx(nx3, bp)
        return carry

    lax.fori_loop(0, NCHUNKS // 4, body, 0)
    pltpu.make_async_copy(x_hbm.at[is0], rows0, gsem0).wait()
    wait_idx(1)
    wait_idx(2)
    plsc.subcore_barrier()
    pltpu.sync_copy(acc_sh.at[pl.ds(r0, ROWS_PER_TILE)],
                    out_hbm.at[c, pl.ds(r0, ROWS_PER_TILE)])


_sc_agg = functools.partial(
    pl.kernel,
    mesh=plsc.VectorSubcoreMesh(core_axis_name="c", subcore_axis_name="s"),
    out_type=jax.ShapeDtypeStruct((NC, N_PAD, D), jnp.float32),
    scratch_types=[
        pltpu.VMEM_SHARED((N_PAD, D), jnp.float32),
        pltpu.VMEM((CHUNK, D), jnp.float32),
        pltpu.VMEM((CHUNK, D), jnp.float32),
        pltpu.VMEM((CHUNK,), jnp.int32),
        pltpu.VMEM((CHUNK,), jnp.int32),
        pltpu.VMEM((CHUNK,), jnp.int32),
        pltpu.VMEM((CHUNK,), jnp.int32),
        pltpu.VMEM((CHUNK,), jnp.int32),
        pltpu.VMEM((CHUNK,), jnp.int32),
        pltpu.VMEM((CHUNK,), jnp.int32),
        pltpu.VMEM((CHUNK,), jnp.int32),
        pltpu.SemaphoreType.DMA,
        pltpu.SemaphoreType.DMA,
        pltpu.SemaphoreType.DMA,
        pltpu.SemaphoreType.DMA,
        pltpu.SemaphoreType.DMA,
        pltpu.SemaphoreType.DMA,
    ],
)(_sc_agg_body)


BLK = 1024


def _tc_linear_body(p_ref, w_ref, b_ref, o_ref):
    agg = p_ref[0] + p_ref[1]
    o_ref[...] = lax.dot_general(
        agg, w_ref[...], (((1,), (1,)), ((), ())),
        preferred_element_type=jnp.float32) + b_ref[...]


def _tc_linear(partials, W, b):
    return pl.pallas_call(
        _tc_linear_body,
        grid=(N_PAD // BLK,),
        in_specs=[
            pl.BlockSpec((NC, BLK, D), lambda i: (0, i, 0)),
            pl.BlockSpec((D, D), lambda i: (0, 0)),
            pl.BlockSpec((1, D), lambda i: (0, 0)),
        ],
        out_specs=pl.BlockSpec((BLK, D), lambda i: (i, 0)),
        out_shape=jax.ShapeDtypeStruct((N_PAD, D), jnp.float32),
    )(partials, W, b.reshape(1, D))


def kernel(x, edge_index, W, b):
    src = edge_index[0].astype(jnp.int32)
    dst = edge_index[1].astype(jnp.int32)
    pad_src = jnp.zeros((NW, PAD_TILE), jnp.int32)
    pad_dst = jnp.broadcast_to(
        N_NODES + jnp.arange(PAD_TILE, dtype=jnp.int32), (NW, PAD_TILE))
    src3 = jnp.concatenate([src.reshape(NW, E_TILE), pad_src],
                           axis=1).reshape(NW, NCHUNKS, CHUNK)
    dst3 = jnp.concatenate([dst.reshape(NW, E_TILE), pad_dst],
                           axis=1).reshape(NW, NCHUNKS, CHUNK)
    zero = jnp.zeros((N_PAD, D), jnp.float32)
    partials = _sc_agg(x, src3, dst3, zero)
    return _tc_linear(partials, W, b)[:N_NODES]


# speedup vs baseline: 2.5638x; 2.5638x over previous
import functools

import jax
import jax.numpy as jnp
from jax import lax
from jax.experimental import pallas as pl
from jax.experimental.pallas import tpu as pltpu
from jax.experimental.pallas import tpu_sc as plsc

N_NODES = 10000
N_PAD = 10240
D = 128
N_EDGES = 320000
NC = 2
NS = 16
EDGES_PER_TILE = N_EDGES // (NC * NS)
CHUNK = 80
NCHUNKS = EDGES_PER_TILE // CHUNK
ROWS_PER_TILE = N_PAD // NS


def _sc_agg_body(x_hbm, ei_hbm, zero_hbm, out_hbm,
                 acc_sh, is0, is1, id0, id1, rows0, rows1, gsem0, gsem1):
    c = lax.axis_index("c")
    s = lax.axis_index("s")
    r0 = s * ROWS_PER_TILE
    pltpu.sync_copy(zero_hbm.at[pl.ds(r0, ROWS_PER_TILE)],
                    acc_sh.at[pl.ds(r0, ROWS_PER_TILE)])
    plsc.subcore_barrier()

    base = (c * NS + s) * EDGES_PER_TILE
    isb = (is0, is1)
    idb = (id0, id1)
    rows = (rows0, rows1)
    gsem = (gsem0, gsem1)

    pltpu.sync_copy(ei_hbm.at[pl.ds(base, CHUNK)], is0)
    pltpu.sync_copy(ei_hbm.at[pl.ds(N_EDGES + base, CHUNK)], id0)
    pltpu.async_copy(x_hbm.at[is0], rows0, gsem0)

    def body(t, carry):
        for b in range(2):
            i = 2 * t + b
            bn = 1 - b
            off = base + (i + 1) * CHUNK
            pltpu.sync_copy(ei_hbm.at[pl.ds(off, CHUNK)], isb[bn])
            pltpu.sync_copy(ei_hbm.at[pl.ds(N_EDGES + off, CHUNK)], idb[bn])
            pltpu.async_copy(x_hbm.at[isb[bn]], rows[bn], gsem[bn])
            pltpu.make_async_copy(x_hbm.at[isb[b]], rows[b],
                                  gsem[b]).wait()
            pltpu.sync_copy(rows[b], acc_sh.at[idb[b]], add=True)
        return carry

    lax.fori_loop(0, (NCHUNKS - 1) // 2, body, 0)
    pltpu.make_async_copy(x_hbm.at[is0], rows0, gsem0).wait()
    pltpu.sync_copy(rows0, acc_sh.at[id0], add=True)
    plsc.subcore_barrier()
    pltpu.sync_copy(acc_sh.at[pl.ds(r0, ROWS_PER_TILE)],
                    out_hbm.at[c, pl.ds(r0, ROWS_PER_TILE)])


_sc_agg = functools.partial(
    pl.kernel,
    mesh=plsc.VectorSubcoreMesh(core_axis_name="c", subcore_axis_name="s"),
    out_type=jax.ShapeDtypeStruct((NC, N_PAD, D), jnp.float32),
    scratch_types=[
        pltpu.VMEM_SHARED((N_PAD, D), jnp.float32),
        pltpu.VMEM((CHUNK,), jnp.int32),
        pltpu.VMEM((CHUNK,), jnp.int32),
        pltpu.VMEM((CHUNK,), jnp.int32),
        pltpu.VMEM((CHUNK,), jnp.int32),
        pltpu.VMEM((CHUNK, D), jnp.float32),
        pltpu.VMEM((CHUNK, D), jnp.float32),
        pltpu.SemaphoreType.DMA,
        pltpu.SemaphoreType.DMA,
    ],
)(_sc_agg_body)


BLK = 1000


def _tc_linear_body(p_ref, w_ref, b_ref, o_ref):
    agg = p_ref[0] + p_ref[1]
    o_ref[...] = lax.dot_general(
        agg, w_ref[...], (((1,), (1,)), ((), ())),
        preferred_element_type=jnp.float32) + b_ref[...]


def _tc_linear(partials, W, b):
    return pl.pallas_call(
        _tc_linear_body,
        grid=(N_NODES // BLK,),
        in_specs=[
            pl.BlockSpec((NC, BLK, D), lambda i: (0, i, 0)),
            pl.BlockSpec((D, D), lambda i: (0, 0)),
            pl.BlockSpec((1, D), lambda i: (0, 0)),
        ],
        out_specs=pl.BlockSpec((BLK, D), lambda i: (i, 0)),
        out_shape=jax.ShapeDtypeStruct((N_NODES, D), jnp.float32),
    )(partials, W, b.reshape(1, D))


def kernel(x, edge_index, W, b):
    ei = edge_index.astype(jnp.int32).reshape(2 * N_EDGES)
    zero = jnp.zeros((N_PAD, D), jnp.float32)
    partials = _sc_agg(x, ei, zero)
    return _tc_linear(partials, W, b)


# speedup vs baseline: 2.5643x; 1.0002x over previous
import functools

import jax
import jax.numpy as jnp
from jax import lax
from jax.experimental import pallas as pl
from jax.experimental.pallas import tpu as pltpu
from jax.experimental.pallas import tpu_sc as plsc

N_NODES = 10000
N_PAD = 10240
D = 128
N_EDGES = 320000
NC = 2
NS = 16
EDGES_PER_TILE = N_EDGES // (NC * NS)
CHUNK = 80
NCHUNKS = EDGES_PER_TILE // CHUNK
ROWS_PER_TILE = N_PAD // NS


def _sc_agg_body(x_hbm, ei_hbm, zero_hbm, out_hbm,
                 acc_sh, is0, is1, is2, id0, id1, id2,
                 rows0, rows1, rows2, gsem0, gsem1, gsem2):
    c = lax.axis_index("c")
    s = lax.axis_index("s")
    r0 = s * ROWS_PER_TILE
    pltpu.sync_copy(zero_hbm.at[pl.ds(r0, ROWS_PER_TILE)],
                    acc_sh.at[pl.ds(r0, ROWS_PER_TILE)])
    plsc.subcore_barrier()

    base = (c * NS + s) * EDGES_PER_TILE
    isb = (is0, is1, is2)
    idb = (id0, id1, id2)
    rows = (rows0, rows1, rows2)
    gsem = (gsem0, gsem1, gsem2)

    def load_idx(n, k):
        off = base + n * CHUNK
        pltpu.sync_copy(ei_hbm.at[pl.ds(off, CHUNK)], isb[k])
        pltpu.sync_copy(ei_hbm.at[pl.ds(N_EDGES + off, CHUNK)], idb[k])

    def finish_chunk(b):
        pltpu.make_async_copy(x_hbm.at[isb[b]], rows[b], gsem[b]).wait()
        pltpu.sync_copy(rows[b], acc_sh.at[idb[b]], add=True)

    load_idx(0, 0)
    load_idx(1, 1)
    pltpu.async_copy(x_hbm.at[is0], rows0, gsem0)
    pltpu.async_copy(x_hbm.at[is1], rows1, gsem1)

    def body(t, carry):
        for b in range(3):
            i = 3 * t + b
            b2 = (b + 2) % 3
            load_idx(i + 2, b2)
            pltpu.async_copy(x_hbm.at[isb[b2]], rows[b2], gsem[b2])
            finish_chunk(b)
        return carry

    lax.fori_loop(0, (NCHUNKS - 2) // 3, body, 0)
    finish_chunk(0)
    finish_chunk(1)
    plsc.subcore_barrier()
    pltpu.sync_copy(acc_sh.at[pl.ds(r0, ROWS_PER_TILE)],
                    out_hbm.at[c, pl.ds(r0, ROWS_PER_TILE)])


_sc_agg = functools.partial(
    pl.kernel,
    mesh=plsc.VectorSubcoreMesh(core_axis_name="c", subcore_axis_name="s"),
    out_type=jax.ShapeDtypeStruct((NC, N_PAD, D), jnp.float32),
    scratch_types=[
        pltpu.VMEM_SHARED((N_PAD, D), jnp.float32),
        pltpu.VMEM((CHUNK,), jnp.int32),
        pltpu.VMEM((CHUNK,), jnp.int32),
        pltpu.VMEM((CHUNK,), jnp.int32),
        pltpu.VMEM((CHUNK,), jnp.int32),
        pltpu.VMEM((CHUNK,), jnp.int32),
        pltpu.VMEM((CHUNK,), jnp.int32),
        pltpu.VMEM((CHUNK, D), jnp.float32),
        pltpu.VMEM((CHUNK, D), jnp.float32),
        pltpu.VMEM((CHUNK, D), jnp.float32),
        pltpu.SemaphoreType.DMA,
        pltpu.SemaphoreType.DMA,
        pltpu.SemaphoreType.DMA,
    ],
)(_sc_agg_body)


BLK = 1000


def _tc_linear_body(p_ref, w_ref, b_ref, o_ref):
    agg = p_ref[0] + p_ref[1]
    o_ref[...] = lax.dot_general(
        agg, w_ref[...], (((1,), (1,)), ((), ())),
        preferred_element_type=jnp.float32) + b_ref[...]


def _tc_linear(partials, W, b):
    return pl.pallas_call(
        _tc_linear_body,
        grid=(N_NODES // BLK,),
        in_specs=[
            pl.BlockSpec((NC, BLK, D), lambda i: (0, i, 0)),
            pl.BlockSpec((D, D), lambda i: (0, 0)),
            pl.BlockSpec((1, D), lambda i: (0, 0)),
        ],
        out_specs=pl.BlockSpec((BLK, D), lambda i: (i, 0)),
        out_shape=jax.ShapeDtypeStruct((N_NODES, D), jnp.float32),
    )(partials, W, b.reshape(1, D))


def kernel(x, edge_index, W, b):
    ei = edge_index.astype(jnp.int32).reshape(2 * N_EDGES)
    zero = jnp.zeros((N_PAD, D), jnp.float32)
    partials = _sc_agg(x, ei, zero)
    return _tc_linear(partials, W, b)


# speedup vs baseline: 2.5646x; 1.0001x over previous
import functools

import jax
import jax.numpy as jnp
from jax import lax
from jax.experimental import pallas as pl
from jax.experimental.pallas import tpu as pltpu
from jax.experimental.pallas import tpu_sc as plsc

N_NODES = 10000
N_PAD = 10240
D = 128
N_EDGES = 320000
NC = 2
NS = 16
EDGES_PER_TILE = N_EDGES // (NC * NS)
CHUNK = 80
NCHUNKS = EDGES_PER_TILE // CHUNK
ROWS_PER_TILE = N_PAD // NS


def _sc_agg_body(x_hbm, ei_hbm, zero_hbm, out_hbm,
                 acc_sh, is0, is1, id0, id1, rows0, rows1, gsem0, gsem1):
    c = lax.axis_index("c")
    s = lax.axis_index("s")
    r0 = s * ROWS_PER_TILE
    pltpu.sync_copy(zero_hbm.at[pl.ds(r0, ROWS_PER_TILE)],
                    acc_sh.at[pl.ds(r0, ROWS_PER_TILE)])
    plsc.subcore_barrier()

    base = (c * NS + s) * EDGES_PER_TILE
    isb = (is0, is1)
    idb = (id0, id1)
    rows = (rows0, rows1)
    gsem = (gsem0, gsem1)

    pltpu.sync_copy(ei_hbm.at[pl.ds(base, CHUNK)], is0)
    pltpu.sync_copy(ei_hbm.at[pl.ds(N_EDGES + base, CHUNK)], id0)
    pltpu.async_copy(x_hbm.at[is0], rows0, gsem0)

    def body(t, carry):
        for b in range(2):
            i = 2 * t + b
            bn = 1 - b
            off = base + (i + 1) * CHUNK
            pltpu.sync_copy(ei_hbm.at[pl.ds(off, CHUNK)], isb[bn])
            pltpu.sync_copy(ei_hbm.at[pl.ds(N_EDGES + off, CHUNK)], idb[bn])
            pltpu.async_copy(x_hbm.at[isb[bn]], rows[bn], gsem[bn])
            pltpu.make_async_copy(x_hbm.at[isb[b]], rows[b],
                                  gsem[b]).wait()
            pltpu.sync_copy(rows[b], acc_sh.at[idb[b]], add=True)
        return carry

    lax.fori_loop(0, (NCHUNKS - 1) // 2, body, 0)
    pltpu.make_async_copy(x_hbm.at[is0], rows0, gsem0).wait()
    pltpu.sync_copy(rows0, acc_sh.at[id0], add=True)
    plsc.subcore_barrier()
    pltpu.sync_copy(acc_sh.at[pl.ds(r0, ROWS_PER_TILE)],
                    out_hbm.at[c, pl.ds(r0, ROWS_PER_TILE)])


_sc_agg = functools.partial(
    pl.kernel,
    mesh=plsc.VectorSubcoreMesh(core_axis_name="c", subcore_axis_name="s"),
    out_type=jax.ShapeDtypeStruct((NC, N_PAD, D), jnp.float32),
    scratch_types=[
        pltpu.VMEM_SHARED((N_PAD, D), jnp.float32),
        pltpu.VMEM((CHUNK,), jnp.int32),
        pltpu.VMEM((CHUNK,), jnp.int32),
        pltpu.VMEM((CHUNK,), jnp.int32),
        pltpu.VMEM((CHUNK,), jnp.int32),
        pltpu.VMEM((CHUNK, D), jnp.float32),
        pltpu.VMEM((CHUNK, D), jnp.float32),
        pltpu.SemaphoreType.DMA,
        pltpu.SemaphoreType.DMA,
    ],
)(_sc_agg_body)


BLK = 1000


def _tc_linear_body(p_ref, w_ref, b_ref, o_ref):
    agg = p_ref[0] + p_ref[1]
    o_ref[...] = lax.dot_general(
        agg, w_ref[...], (((1,), (1,)), ((), ())),
        preferred_element_type=jnp.float32) + b_ref[...]


def _tc_linear(partials, W, b):
    return pl.pallas_call(
        _tc_linear_body,
        grid=(N_NODES // BLK,),
        in_specs=[
            pl.BlockSpec((NC, BLK, D), lambda i: (0, i, 0)),
            pl.BlockSpec((D, D), lambda i: (0, 0)),
            pl.BlockSpec((1, D), lambda i: (0, 0)),
        ],
        out_specs=pl.BlockSpec((BLK, D), lambda i: (i, 0)),
        out_shape=jax.ShapeDtypeStruct((N_NODES, D), jnp.float32),
    )(partials, W, b.reshape(1, D))


def kernel(x, edge_index, W, b):
    ei = edge_index.astype(jnp.int32).reshape(2 * N_EDGES)
    zero = jnp.zeros((N_PAD, D), jnp.float32)
    partials = _sc_agg(x, ei, zero)
    return _tc_linear(partials, W, b)
